# baseline (speedup 1.0000x reference)
import jax
import jax.numpy as jnp
from jax.experimental import pallas as pl
from jax.experimental.pallas import tpu as pltpu

_TM = 2048
_TN = 2048
_TK = 512


def _mm_kernel(a_ref, b_ref, o_ref):
    @pl.when(pl.program_id(2) == 0)
    def _():
        o_ref[...] = jnp.zeros_like(o_ref)

    o_ref[...] += jnp.dot(
        a_ref[...].astype(jnp.bfloat16),
        b_ref[...].astype(jnp.bfloat16),
        preferred_element_type=jnp.float32,
    )


def kernel(a, b):
    M, K = a.shape
    K2, N = b.shape
    assert K == K2

    grid_m = -(-M // _TM)
    grid_n = -(-N // _TN)
    grid_k = -(-K // _TK)

    return pl.pallas_call(
        _mm_kernel,
        out_shape=jax.ShapeDtypeStruct((M, N), jnp.float32),
        grid=(grid_n, grid_m, grid_k),
        in_specs=[
            pl.BlockSpec((_TM, _TK), lambda j, i, k: (i, k)),
            pl.BlockSpec((_TK, _TN), lambda j, i, k: (k, j)),
        ],
        out_specs=pl.BlockSpec((_TM, _TN), lambda j, i, k: (i, j)),
        compiler_params=pltpu.CompilerParams(
            dimension_semantics=("parallel", "parallel", "arbitrary"),
            vmem_limit_bytes=59392 * 1024,
        ),
        cost_estimate=pl.CostEstimate(
            flops=2 * M * N * K,
            transcendentals=0,
            bytes_accessed=(2 * M * K + 2 * K * N + M * N) * 4,
        ),
    )(a, b)


# speedup vs baseline: 1.0004x; 1.0004x over previous
import jax
import jax.numpy as jnp
from jax.experimental import pallas as pl
from jax.experimental.pallas import tpu as pltpu

_TM = 2048
_TN = 2048
_TK = 512


def _mm_kernel(a_ref, b_ref, o_ref):
    @pl.when(pl.program_id(2) == 0)
    def _():
        o_ref[...] = jnp.zeros_like(o_ref)

    o_ref[...] += jnp.dot(
        a_ref[...].astype(jnp.bfloat16),
        b_ref[...].astype(jnp.bfloat16),
        preferred_element_type=jnp.float32,
    )


def kernel(a, b):
    M, K = a.shape
    K2, N = b.shape
    assert K == K2

    grid_m = -(-M // _TM)
    grid_n = -(-N // _TN)
    grid_k = -(-K // _TK)

    return pl.pallas_call(
        _mm_kernel,
        out_shape=jax.ShapeDtypeStruct((M, N), jnp.float32),
        grid=(grid_m, grid_n, grid_k),
        in_specs=[
            pl.BlockSpec((_TM, _TK), lambda i, j, k: (i, k)),
            pl.BlockSpec((_TK, _TN), lambda i, j, k: (k, j)),
        ],
        out_specs=pl.BlockSpec((_TM, _TN), lambda i, j, k: (i, j)),
        compiler_params=pltpu.CompilerParams(
            dimension_semantics=("parallel", "parallel", "arbitrary"),
            vmem_limit_bytes=59392 * 1024,
        ),
        cost_estimate=pl.CostEstimate(
            flops=2 * M * N * K,
            transcendentals=0,
            bytes_accessed=(2 * M * K + 2 * K * N + M * N) * 4,
        ),
    )(a, b)
